# baseline (speedup 1.0000x reference)
import jax
import jax.numpy as jnp
from jax import lax
from jax.experimental import pallas as pl
from jax.experimental.pallas import tpu as pltpu
from jax.experimental.pallas import tpu_sc as plsc

N = 10000
E = 320000
D = 128
H = 64
NC = 2
NS = 16
NW = NC * NS
L = 16
NPAD = 10240
EB = 80
NB_DEG = (E // NS) // EB
NB_EDGE = (E // NW) // EB
VPT = NPAD // NS
OCH = 128


def _sc_body(src4_hbm, dst4_hbm, x_hbm,
             aggx_out, c_out, deg_out,
             srcbuf, dst2buf, rowbuf,
             vals, ones_v, zvec, degbuf, rdegbuf,
             sem_deg, sem_ga0, sem_ga1, sem_sa0, sem_sa1,
             sem_gb0, sem_gb1, sem_sb0, sem_sb1,
             deg_s, c_s, rdeg_s, aggx_s):
    cid = lax.axis_index("c")
    sid = lax.axis_index("s")
    wid = cid * NS + sid
    sem_ga = (sem_ga0, sem_ga1)
    sem_sa = (sem_sa0, sem_sa1)
    sem_gb = (sem_gb0, sem_gb1)
    sem_sb = (sem_sb0, sem_sb1)

    zero16 = jnp.zeros((L,), jnp.float32)
    one16 = jnp.ones((L,), jnp.float32)

    def _zvec_step(i, carry):
        zvec[pl.ds(i * L, L)] = zero16
        return carry
    lax.fori_loop(0, VPT // L, _zvec_step, 0)

    def _zrow_step(i, carry):
        for ch in range(D // L):
            rowbuf[0, i, pl.ds(ch * L, L)] = zero16
        return carry
    lax.fori_loop(0, EB, _zrow_step, 0)

    for i in range(EB // L):
        ones_v[pl.ds(i * L, L)] = one16

    pltpu.sync_copy(zvec, deg_s.at[pl.ds(sid * VPT, VPT)])
    pltpu.sync_copy(zvec, c_s.at[pl.ds(sid * VPT, VPT)])
    for i in range(VPT // EB):
        pltpu.sync_copy(rowbuf.at[0],
                        aggx_s.at[pl.ds(sid * VPT + i * EB, EB)])
    plsc.subcore_barrier()

    GRP = 25

    def _deg_grp(g, carry):
        def _fire(k, c2):
            pltpu.async_copy(ones_v, deg_s.at[dst2buf.at[g * GRP + k]],
                             sem_deg, add=True)
            return c2
        lax.fori_loop(0, GRP, _fire, 0)

        def _drain(k, c2):
            pltpu.make_async_copy(ones_v, deg_s.at[dst2buf.at[g * GRP + k]],
                                  sem_deg).wait()
            return c2
        lax.fori_loop(0, GRP, _drain, 0)
        return carry

    for half in range(NB_DEG // NB_EDGE):
        pltpu.sync_copy(dst4_hbm.at[sid, pl.ds(half * NB_EDGE, NB_EDGE)],
                        dst2buf)
        lax.fori_loop(0, NB_EDGE // GRP, _deg_grp, 0)
    plsc.subcore_barrier()

    pltpu.sync_copy(deg_s.at[pl.ds(sid * VPT, VPT)], degbuf)
    for i in range(VPT // L):
        v = degbuf[pl.ds(i * L, L)]
        rdegbuf[pl.ds(i * L, L)] = 1.0 / jnp.maximum(v, 1.0)
    pltpu.sync_copy(rdegbuf, rdeg_s.at[pl.ds(sid * VPT, VPT)])
    pltpu.sync_copy(degbuf, deg_out.at[cid, 0, pl.ds(sid * VPT, VPT)])
    plsc.subcore_barrier()

    erow = wid // 2
    eoff = (wid % 2) * NB_EDGE
    pltpu.sync_copy(src4_hbm.at[erow, pl.ds(eoff, NB_EDGE)], srcbuf)
    pltpu.sync_copy(dst4_hbm.at[erow, pl.ds(eoff, NB_EDGE)], dst2buf)

    def _start(j, s):
        pltpu.async_copy(rdeg_s.at[dst2buf.at[j]], vals.at[s], sem_ga[s])
        pltpu.async_copy(x_hbm.at[srcbuf.at[j]], rowbuf.at[s], sem_gb[s])

    def _finish(j, s):
        pltpu.make_async_copy(rdeg_s.at[dst2buf.at[j]], vals.at[s],
                              sem_ga[s]).wait()
        pltpu.make_async_copy(x_hbm.at[srcbuf.at[j]], rowbuf.at[s],
                              sem_gb[s]).wait()
        pltpu.async_copy(vals.at[s], c_s.at[srcbuf.at[j]], sem_sa[s],
                         add=True)
        pltpu.async_copy(rowbuf.at[s], aggx_s.at[dst2buf.at[j]], sem_sb[s],
                         add=True)

    def _drain(j, s):
        pltpu.make_async_copy(vals.at[s], c_s.at[srcbuf.at[j]],
                              sem_sa[s]).wait()
        pltpu.make_async_copy(rowbuf.at[s], aggx_s.at[dst2buf.at[j]],
                              sem_sb[s]).wait()

    _start(0, 0)
    _start(1, 1)

    def _pipe(i, carry):
        j0 = 2 * i
        _finish(j0, 0)
        _drain(j0, 0)
        _start(j0 + 2, 0)
        j1 = j0 + 1
        _finish(j1, 1)
        _drain(j1, 1)
        _start(j1 + 2, 1)
        return carry
    lax.fori_loop(0, (NB_EDGE - 3) // 2, _pipe, 0)
    _finish(NB_EDGE - 3, 0)
    _drain(NB_EDGE - 3, 0)
    _start(NB_EDGE - 1, 0)
    _finish(NB_EDGE - 2, 1)
    _drain(NB_EDGE - 2, 1)
    _finish(NB_EDGE - 1, 0)
    _drain(NB_EDGE - 1, 0)
    plsc.subcore_barrier()

    pltpu.sync_copy(c_s.at[pl.ds(sid * VPT, VPT)], degbuf)
    pltpu.sync_copy(degbuf, c_out.at[cid, 0, pl.ds(sid * VPT, VPT)])
    for k in range(VPT // EB):
        r0 = sid * VPT + k * EB
        s = k % 2
        pltpu.sync_copy(aggx_s.at[pl.ds(r0, EB)], rowbuf.at[s])
        pltpu.sync_copy(rowbuf.at[s], aggx_out.at[cid, pl.ds(r0, EB)])


_sc_call = pl.kernel(
    _sc_body,
    out_type=[
        jax.ShapeDtypeStruct((NC, NPAD, D), jnp.float32),
        jax.ShapeDtypeStruct((NC, 1, NPAD), jnp.float32),
        jax.ShapeDtypeStruct((NC, 1, NPAD), jnp.float32),
    ],
    mesh=plsc.VectorSubcoreMesh(core_axis_name="c", subcore_axis_name="s"),
    compiler_params=pltpu.CompilerParams(use_tc_tiling_on_sc=False),
    scratch_types=[
        pltpu.VMEM((NB_EDGE, EB), jnp.int32),
        pltpu.VMEM((NB_EDGE, EB), jnp.int32),
        pltpu.VMEM((2, EB, D), jnp.float32),
        pltpu.VMEM((2, EB), jnp.float32),
        pltpu.VMEM((EB,), jnp.float32),
        pltpu.VMEM((VPT,), jnp.float32),
        pltpu.VMEM((VPT,), jnp.float32),
        pltpu.VMEM((VPT,), jnp.float32),
        pltpu.SemaphoreType.DMA,
        pltpu.SemaphoreType.DMA,
        pltpu.SemaphoreType.DMA,
        pltpu.SemaphoreType.DMA,
        pltpu.SemaphoreType.DMA,
        pltpu.SemaphoreType.DMA,
        pltpu.SemaphoreType.DMA,
        pltpu.SemaphoreType.DMA,
        pltpu.SemaphoreType.DMA,
        pltpu.VMEM_SHARED((NPAD,), jnp.float32),
        pltpu.VMEM_SHARED((NPAD,), jnp.float32),
        pltpu.VMEM_SHARED((NPAD,), jnp.float32),
        pltpu.VMEM_SHARED((NPAD, D), jnp.float32),
    ],
)

_HI = jax.lax.Precision.HIGHEST


NBLK = 10
BR = N // NBLK


def _final_body(x_ref, aggx_ref, deg_ref, c_ref,
                w1s_ref, w1n_ref, b1_ref,
                w2s_ref, w2n_ref, b2_ref, wc_ref, bc_ref, out_ref,
                s1_ref, s2_ref):
    i = pl.program_id(0)
    deg = jnp.maximum(deg_ref[0, :, 0], 1.0)
    agg1 = (aggx_ref[0] + aggx_ref[1]) / deg[:, None]
    xb = x_ref[...]
    h1 = jnp.maximum(
        jnp.dot(xb.astype(jnp.bfloat16), w1s_ref[...].astype(jnp.bfloat16),
                preferred_element_type=jnp.float32)
        + jnp.dot(agg1.astype(jnp.bfloat16),
                  w1n_ref[...].astype(jnp.bfloat16),
                  preferred_element_type=jnp.float32)
        + b1_ref[...], 0.0)
    c = c_ref[0, :, 0] + c_ref[1, :, 0]
    s1 = jnp.sum(h1, axis=0, keepdims=True)
    s2 = jnp.dot(c[None, :], h1, precision=_HI,
                 preferred_element_type=jnp.float32)

    @pl.when(i == 0)
    def _init():
        s1_ref[...] = s1
        s2_ref[...] = s2

    @pl.when(i > 0)
    def _acc():
        s1_ref[...] += s1
        s2_ref[...] += s2

    @pl.when(i == NBLK - 1)
    def _fin():
        w2s = w2s_ref[...].astype(jnp.bfloat16).astype(jnp.float32)
        w2n = w2n_ref[...].astype(jnp.bfloat16).astype(jnp.float32)
        m = (jnp.dot(s1_ref[...] / N, w2s, precision=_HI,
                     preferred_element_type=jnp.float32)
             + jnp.dot(s2_ref[...] / N, w2n, precision=_HI,
                       preferred_element_type=jnp.float32)
             + b2_ref[...])
        out_ref[...] = jnp.dot(m, wc_ref[...], precision=_HI,
                               preferred_element_type=jnp.float32) + bc_ref[...]


def kernel(x, edge_index, W1_self, W1_neigh, b1, W2_self, W2_neigh, b2, Wc, bc):
    src4 = edge_index[0].reshape(NS, NB_DEG, EB)
    dst4 = edge_index[1].reshape(NS, NB_DEG, EB)

    aggx, c, deg = _sc_call(src4, dst4, x)

    full = lambda *shape: pl.BlockSpec(shape, lambda i: (0,) * len(shape))
    pred = pl.pallas_call(
        _final_body,
        grid=(NBLK,),
        in_specs=[
            pl.BlockSpec((BR, D), lambda i: (i, 0)),
            pl.BlockSpec((NC, BR, D), lambda i: (0, i, 0)),
            pl.BlockSpec((NC, BR, 1), lambda i: (0, i, 0)),
            pl.BlockSpec((NC, BR, 1), lambda i: (0, i, 0)),
            full(D, H), full(D, H), full(1, H),
            full(H, H), full(H, H), full(1, H), full(H, 1), full(1, 1),
        ],
        out_specs=pl.BlockSpec((1, 1), lambda i: (0, 0)),
        out_shape=jax.ShapeDtypeStruct((1, 1), jnp.float32),
        scratch_shapes=[
            pltpu.VMEM((1, H), jnp.float32),
            pltpu.VMEM((1, H), jnp.float32),
        ],
    )(x, aggx, deg.reshape(NC, NPAD, 1), c.reshape(NC, NPAD, 1),
      W1_self, W1_neigh, b1.reshape(1, H),
      W2_self, W2_neigh, b2.reshape(1, H), Wc, bc.reshape(1, 1))
    return pred.reshape(1)


# speedup vs baseline: 1.3181x; 1.3181x over previous
import jax
import jax.numpy as jnp
from jax import lax
from jax.experimental import pallas as pl
from jax.experimental.pallas import tpu as pltpu
from jax.experimental.pallas import tpu_sc as plsc

N = 10000
E = 320000
D = 128
H = 64
NC = 2
NS = 16
NW = NC * NS
L = 16
NPAD = 10240
EB = 40
NB_DEG = (E // NS) // EB
NB_EDGE = (E // NW) // EB
VPT = NPAD // NS
NSLOT = 5


def _sc_body(src4_hbm, dst4_hbm, x_hbm,
             aggx_out, c_out, deg_out,
             srcbuf, dst2buf, rowbuf,
             vals, ones_v, degbuf,
             sem_deg,
             sem_ga0, sem_ga1, sem_ga2, sem_ga3, sem_ga4,
             sem_sa0, sem_sa1, sem_sa2, sem_sa3, sem_sa4,
             sem_gb0, sem_gb1, sem_gb2, sem_gb3, sem_gb4,
             sem_sb0, sem_sb1, sem_sb2, sem_sb3, sem_sb4,
             deg_s, c_s, rdeg_s, aggx_s):
    cid = lax.axis_index("c")
    sid = lax.axis_index("s")
    wid = cid * NS + sid
    sem_ga = (sem_ga0, sem_ga1, sem_ga2, sem_ga3, sem_ga4)
    sem_sa = (sem_sa0, sem_sa1, sem_sa2, sem_sa3, sem_sa4)
    sem_gb = (sem_gb0, sem_gb1, sem_gb2, sem_gb3, sem_gb4)
    sem_sb = (sem_sb0, sem_sb1, sem_sb2, sem_sb3, sem_sb4)

    zero16 = jnp.zeros((L,), jnp.float32)
    one16 = jnp.ones((L,), jnp.float32)

    def _zvec_step(i, carry):
        degbuf[pl.ds(i * L, L)] = zero16
        return carry
    lax.fori_loop(0, VPT // L, _zvec_step, 0)

    def _zrow_step(i, carry):
        for ch in range(D // L):
            rowbuf[0, i, pl.ds(ch * L, L)] = zero16
        return carry
    lax.fori_loop(0, EB, _zrow_step, 0)

    for i in range(EB // L):
        ones_v[pl.ds(i * L, L)] = one16

    pltpu.sync_copy(degbuf, deg_s.at[pl.ds(sid * VPT, VPT)])
    pltpu.sync_copy(degbuf, c_s.at[pl.ds(sid * VPT, VPT)])
    for i in range(VPT // EB):
        pltpu.sync_copy(rowbuf.at[0],
                        aggx_s.at[pl.ds(sid * VPT + i * EB, EB)])
    plsc.subcore_barrier()

    GRP = 50

    def _deg_grp(g, carry):
        def _fire(k, c2):
            pltpu.async_copy(ones_v, deg_s.at[dst2buf.at[g * GRP + k]],
                             sem_deg, add=True)
            return c2
        lax.fori_loop(0, GRP, _fire, 0)

        def _draing(k, c2):
            pltpu.make_async_copy(ones_v, deg_s.at[dst2buf.at[g * GRP + k]],
                                  sem_deg).wait()
            return c2
        lax.fori_loop(0, GRP, _draing, 0)
        return carry

    for half in range(NB_DEG // NB_EDGE):
        pltpu.sync_copy(dst4_hbm.at[sid, pl.ds(half * NB_EDGE, NB_EDGE)],
                        dst2buf)
        lax.fori_loop(0, NB_EDGE // GRP, _deg_grp, 0)
    plsc.subcore_barrier()

    pltpu.sync_copy(deg_s.at[pl.ds(sid * VPT, VPT)], degbuf)
    pltpu.sync_copy(degbuf, deg_out.at[cid, 0, pl.ds(sid * VPT, VPT)])
    for i in range(VPT // L):
        v = degbuf[pl.ds(i * L, L)]
        degbuf[pl.ds(i * L, L)] = 1.0 / jnp.maximum(v, 1.0)
    pltpu.sync_copy(degbuf, rdeg_s.at[pl.ds(sid * VPT, VPT)])
    plsc.subcore_barrier()

    erow = wid // 2
    eoff = (wid % 2) * NB_EDGE
    pltpu.sync_copy(src4_hbm.at[erow, pl.ds(eoff, NB_EDGE)], srcbuf)
    pltpu.sync_copy(dst4_hbm.at[erow, pl.ds(eoff, NB_EDGE)], dst2buf)

    def _start(j, s):
        pltpu.async_copy(rdeg_s.at[dst2buf.at[j]], vals.at[s], sem_ga[s])
        pltpu.async_copy(x_hbm.at[srcbuf.at[j]], rowbuf.at[s], sem_gb[s])

    def _finish(j, s):
        pltpu.make_async_copy(rdeg_s.at[dst2buf.at[j]], vals.at[s],
                              sem_ga[s]).wait()
        pltpu.make_async_copy(x_hbm.at[srcbuf.at[j]], rowbuf.at[s],
                              sem_gb[s]).wait()
        pltpu.async_copy(vals.at[s], c_s.at[srcbuf.at[j]], sem_sa[s],
                         add=True)
        pltpu.async_copy(rowbuf.at[s], aggx_s.at[dst2buf.at[j]], sem_sb[s],
                         add=True)

    def _drain(j, s):
        pltpu.make_async_copy(vals.at[s], c_s.at[srcbuf.at[j]],
                              sem_sa[s]).wait()
        pltpu.make_async_copy(rowbuf.at[s], aggx_s.at[dst2buf.at[j]],
                              sem_sb[s]).wait()

    for s in range(NSLOT):
        _start(s, s)

    def _pipe(g, carry):
        j0 = NSLOT * g
        for s in range(NSLOT):
            _finish(j0 + s, s)
            _drain(j0 + s, s)
            _start(j0 + s + NSLOT, s)
        return carry
    lax.fori_loop(0, NB_EDGE // NSLOT - 1, _pipe, 0)
    for s in range(NSLOT):
        _finish(NB_EDGE - NSLOT + s, s)
        _drain(NB_EDGE - NSLOT + s, s)
    plsc.subcore_barrier()

    pltpu.sync_copy(c_s.at[pl.ds(sid * VPT, VPT)], degbuf)
    pltpu.sync_copy(degbuf, c_out.at[cid, 0, pl.ds(sid * VPT, VPT)])
    for k in range(VPT // EB):
        r0 = sid * VPT + k * EB
        s = k % NSLOT
        pltpu.sync_copy(aggx_s.at[pl.ds(r0, EB)], rowbuf.at[s])
        pltpu.sync_copy(rowbuf.at[s], aggx_out.at[cid, pl.ds(r0, EB)])


_sc_call = pl.kernel(
    _sc_body,
    out_type=[
        jax.ShapeDtypeStruct((NC, NPAD, D), jnp.float32),
        jax.ShapeDtypeStruct((NC, 1, NPAD), jnp.float32),
        jax.ShapeDtypeStruct((NC, 1, NPAD), jnp.float32),
    ],
    mesh=plsc.VectorSubcoreMesh(core_axis_name="c", subcore_axis_name="s"),
    compiler_params=pltpu.CompilerParams(use_tc_tiling_on_sc=False),
    scratch_types=[
        pltpu.VMEM((NB_EDGE, EB), jnp.int32),
        pltpu.VMEM((NB_EDGE, EB), jnp.int32),
        pltpu.VMEM((NSLOT, EB, D), jnp.float32),
        pltpu.VMEM((NSLOT, EB), jnp.float32),
        pltpu.VMEM((EB,), jnp.float32),
        pltpu.VMEM((VPT,), jnp.float32),
    ] + [pltpu.SemaphoreType.DMA] * 21 + [
        pltpu.VMEM_SHARED((NPAD,), jnp.float32),
        pltpu.VMEM_SHARED((NPAD,), jnp.float32),
        pltpu.VMEM_SHARED((NPAD,), jnp.float32),
        pltpu.VMEM_SHARED((NPAD, D), jnp.float32),
    ],
)

_HI = jax.lax.Precision.HIGHEST


def _final_body(x_ref, aggx_ref, deg_ref, c_ref,
                w1s_ref, w1n_ref, b1_ref,
                w2s_ref, w2n_ref, b2_ref, wc_ref, bc_ref, out_ref):
    deg = jnp.maximum(deg_ref[0, 0, :N], 1.0)
    agg1 = (aggx_ref[0, :N] + aggx_ref[1, :N]) / deg[:, None]
    xb = x_ref[...]
    h1 = jnp.maximum(
        jnp.dot(xb.astype(jnp.bfloat16), w1s_ref[...].astype(jnp.bfloat16),
                preferred_element_type=jnp.float32)
        + jnp.dot(agg1.astype(jnp.bfloat16),
                  w1n_ref[...].astype(jnp.bfloat16),
                  preferred_element_type=jnp.float32)
        + b1_ref[...], 0.0)
    c = c_ref[0, 0, :N] + c_ref[1, 0, :N]
    s1 = jnp.sum(h1, axis=0, keepdims=True)
    s2 = jnp.dot(c[None, :], h1, precision=_HI,
                 preferred_element_type=jnp.float32)
    w2s = w2s_ref[...].astype(jnp.bfloat16).astype(jnp.float32)
    w2n = w2n_ref[...].astype(jnp.bfloat16).astype(jnp.float32)
    m = (jnp.dot(s1 / N, w2s, precision=_HI,
                 preferred_element_type=jnp.float32)
         + jnp.dot(s2 / N, w2n, precision=_HI,
                   preferred_element_type=jnp.float32)
         + b2_ref[...])
    out_ref[...] = jnp.dot(m, wc_ref[...], precision=_HI,
                           preferred_element_type=jnp.float32) + bc_ref[...]


def kernel(x, edge_index, W1_self, W1_neigh, b1, W2_self, W2_neigh, b2, Wc, bc):
    src4 = edge_index[0].reshape(NS, NB_DEG, EB)
    dst4 = edge_index[1].reshape(NS, NB_DEG, EB)

    aggx, c, deg = _sc_call(src4, dst4, x)

    pred = pl.pallas_call(
        _final_body,
        out_shape=jax.ShapeDtypeStruct((1, 1), jnp.float32),
    )(x, aggx, deg, c, W1_self, W1_neigh, b1.reshape(1, H),
      W2_self, W2_neigh, b2.reshape(1, H), Wc, bc.reshape(1, 1))
    return pred.reshape(1)


# speedup vs baseline: 1.3185x; 1.0003x over previous
import jax
import jax.numpy as jnp
from jax import lax
from jax.experimental import pallas as pl
from jax.experimental.pallas import tpu as pltpu
from jax.experimental.pallas import tpu_sc as plsc

N = 10000
E = 320000
D = 128
H = 64
NC = 2
NS = 16
NW = NC * NS
L = 16
NPAD = 10240
EB = 40
NB_DEG = (E // NS) // EB
NB_EDGE = (E // NW) // EB
VPT = NPAD // NS
NSLOT = 5


def _sc_body(src4_hbm, dst4_hbm, x_hbm,
             aggx_out, c_out, deg_out,
             srcbuf, dst2buf, rowbuf,
             vals, ones_v, degbuf,
             sem_deg,
             sem_ga0, sem_ga1, sem_ga2, sem_ga3, sem_ga4,
             sem_sa0, sem_sa1, sem_sa2, sem_sa3, sem_sa4,
             sem_gb0, sem_gb1, sem_gb2, sem_gb3, sem_gb4,
             sem_sb0, sem_sb1, sem_sb2, sem_sb3, sem_sb4,
             deg_s, c_s, rdeg_s, aggx_s):
    cid = lax.axis_index("c")
    sid = lax.axis_index("s")
    wid = cid * NS + sid
    sem_ga = (sem_ga0, sem_ga1, sem_ga2, sem_ga3, sem_ga4)
    sem_sa = (sem_sa0, sem_sa1, sem_sa2, sem_sa3, sem_sa4)
    sem_gb = (sem_gb0, sem_gb1, sem_gb2, sem_gb3, sem_gb4)
    sem_sb = (sem_sb0, sem_sb1, sem_sb2, sem_sb3, sem_sb4)

    zero16 = jnp.zeros((L,), jnp.float32)
    one16 = jnp.ones((L,), jnp.float32)

    def _zvec_step(i, carry):
        degbuf[pl.ds(i * L, L)] = zero16
        return carry
    lax.fori_loop(0, VPT // L, _zvec_step, 0)

    def _zrow_step(i, carry):
        for ch in range(D // L):
            rowbuf[0, i, pl.ds(ch * L, L)] = zero16
        return carry
    lax.fori_loop(0, EB, _zrow_step, 0)

    for off in range(0, EB - L + 1, L):
        ones_v[pl.ds(off, L)] = one16
    if EB % L:
        ones_v[pl.ds(EB - L, L)] = one16

    pltpu.sync_copy(degbuf, deg_s.at[pl.ds(sid * VPT, VPT)])
    pltpu.sync_copy(degbuf, c_s.at[pl.ds(sid * VPT, VPT)])
    for i in range(VPT // EB):
        pltpu.sync_copy(rowbuf.at[0],
                        aggx_s.at[pl.ds(sid * VPT + i * EB, EB)])
    plsc.subcore_barrier()

    GRP = 50

    def _deg_grp(g, carry):
        def _fire(k, c2):
            pltpu.async_copy(ones_v, deg_s.at[dst2buf.at[g * GRP + k]],
                             sem_deg, add=True)
            return c2
        lax.fori_loop(0, GRP, _fire, 0)

        def _draing(k, c2):
            pltpu.make_async_copy(ones_v, deg_s.at[dst2buf.at[g * GRP + k]],
                                  sem_deg).wait()
            return c2
        lax.fori_loop(0, GRP, _draing, 0)
        return carry

    for half in range(NB_DEG // NB_EDGE):
        pltpu.sync_copy(dst4_hbm.at[sid, pl.ds(half * NB_EDGE, NB_EDGE)],
                        dst2buf)
        lax.fori_loop(0, NB_EDGE // GRP, _deg_grp, 0)
    plsc.subcore_barrier()

    pltpu.sync_copy(deg_s.at[pl.ds(sid * VPT, VPT)], degbuf)
    pltpu.sync_copy(degbuf, deg_out.at[cid, 0, pl.ds(sid * VPT, VPT)])
    for i in range(VPT // L):
        v = degbuf[pl.ds(i * L, L)]
        degbuf[pl.ds(i * L, L)] = 1.0 / jnp.maximum(v, 1.0)
    pltpu.sync_copy(degbuf, rdeg_s.at[pl.ds(sid * VPT, VPT)])
    plsc.subcore_barrier()

    erow = wid // 2
    eoff = (wid % 2) * NB_EDGE
    pltpu.sync_copy(src4_hbm.at[erow, pl.ds(eoff, NB_EDGE)], srcbuf)
    pltpu.sync_copy(dst4_hbm.at[erow, pl.ds(eoff, NB_EDGE)], dst2buf)

    def _start(j, s):
        pltpu.async_copy(rdeg_s.at[dst2buf.at[j]], vals.at[s], sem_ga[s])
        pltpu.async_copy(x_hbm.at[srcbuf.at[j]], rowbuf.at[s], sem_gb[s])

    def _finish(j, s):
        pltpu.make_async_copy(rdeg_s.at[dst2buf.at[j]], vals.at[s],
                              sem_ga[s]).wait()
        pltpu.make_async_copy(x_hbm.at[srcbuf.at[j]], rowbuf.at[s],
                              sem_gb[s]).wait()
        pltpu.async_copy(vals.at[s], c_s.at[srcbuf.at[j]], sem_sa[s],
                         add=True)
        pltpu.async_copy(rowbuf.at[s], aggx_s.at[dst2buf.at[j]], sem_sb[s],
                         add=True)

    def _drain(j, s):
        pltpu.make_async_copy(vals.at[s], c_s.at[srcbuf.at[j]],
                              sem_sa[s]).wait()
        pltpu.make_async_copy(rowbuf.at[s], aggx_s.at[dst2buf.at[j]],
                              sem_sb[s]).wait()

    for s in range(NSLOT):
        _start(s, s)

    def _pipe(g, carry):
        j0 = NSLOT * g
        for s in range(NSLOT):
            _finish(j0 + s, s)
            _drain(j0 + s, s)
            _start(j0 + s + NSLOT, s)
        return carry
    lax.fori_loop(0, NB_EDGE // NSLOT - 1, _pipe, 0)
    for s in range(NSLOT):
        _finish(NB_EDGE - NSLOT + s, s)
        _drain(NB_EDGE - NSLOT + s, s)
    plsc.subcore_barrier()

    pltpu.sync_copy(c_s.at[pl.ds(sid * VPT, VPT)], degbuf)
    pltpu.sync_copy(degbuf, c_out.at[cid, 0, pl.ds(sid * VPT, VPT)])
    for k in range(VPT // EB):
        r0 = sid * VPT + k * EB
        s = k % NSLOT
        pltpu.sync_copy(aggx_s.at[pl.ds(r0, EB)], rowbuf.at[s])
        pltpu.sync_copy(rowbuf.at[s], aggx_out.at[cid, pl.ds(r0, EB)])


_sc_call = pl.kernel(
    _sc_body,
    out_type=[
        jax.ShapeDtypeStruct((NC, NPAD, D), jnp.float32),
        jax.ShapeDtypeStruct((NC, 1, NPAD), jnp.float32),
        jax.ShapeDtypeStruct((NC, 1, NPAD), jnp.float32),
    ],
    mesh=plsc.VectorSubcoreMesh(core_axis_name="c", subcore_axis_name="s"),
    compiler_params=pltpu.CompilerParams(use_tc_tiling_on_sc=False),
    scratch_types=[
        pltpu.VMEM((NB_EDGE, EB), jnp.int32),
        pltpu.VMEM((NB_EDGE, EB), jnp.int32),
        pltpu.VMEM((NSLOT, EB, D), jnp.float32),
        pltpu.VMEM((NSLOT, EB), jnp.float32),
        pltpu.VMEM((EB,), jnp.float32),
        pltpu.VMEM((VPT,), jnp.float32),
    ] + [pltpu.SemaphoreType.DMA] * 21 + [
        pltpu.VMEM_SHARED((NPAD,), jnp.float32),
        pltpu.VMEM_SHARED((NPAD,), jnp.float32),
        pltpu.VMEM_SHARED((NPAD,), jnp.float32),
        pltpu.VMEM_SHARED((NPAD, D), jnp.float32),
    ],
)

_HI = jax.lax.Precision.HIGHEST


def _final_body(x_ref, aggx_ref, deg_ref, c_ref,
                w1s_ref, w1n_ref, b1_ref,
                w2s_ref, w2n_ref, b2_ref, wc_ref, bc_ref, out_ref):
    deg = jnp.maximum(deg_ref[0, 0, :N], 1.0)
    agg1 = (aggx_ref[0, :N] + aggx_ref[1, :N]) / deg[:, None]
    xb = x_ref[...]
    h1 = jnp.maximum(
        jnp.dot(xb.astype(jnp.bfloat16), w1s_ref[...].astype(jnp.bfloat16),
                preferred_element_type=jnp.float32)
        + jnp.dot(agg1.astype(jnp.bfloat16),
                  w1n_ref[...].astype(jnp.bfloat16),
                  preferred_element_type=jnp.float32)
        + b1_ref[...], 0.0)
    c = c_ref[0, 0, :N] + c_ref[1, 0, :N]
    s1 = jnp.sum(h1, axis=0, keepdims=True)
    s2 = jnp.dot(c[None, :], h1, precision=_HI,
                 preferred_element_type=jnp.float32)
    w2s = w2s_ref[...].astype(jnp.bfloat16).astype(jnp.float32)
    w2n = w2n_ref[...].astype(jnp.bfloat16).astype(jnp.float32)
    m = (jnp.dot(s1 / N, w2s, precision=_HI,
                 preferred_element_type=jnp.float32)
         + jnp.dot(s2 / N, w2n, precision=_HI,
                   preferred_element_type=jnp.float32)
         + b2_ref[...])
    out_ref[...] = jnp.dot(m, wc_ref[...], precision=_HI,
                           preferred_element_type=jnp.float32) + bc_ref[...]


def kernel(x, edge_index, W1_self, W1_neigh, b1, W2_self, W2_neigh, b2, Wc, bc):
    src4 = edge_index[0].reshape(NS, NB_DEG, EB)
    dst4 = edge_index[1].reshape(NS, NB_DEG, EB)

    aggx, c, deg = _sc_call(src4, dst4, x)

    pred = pl.pallas_call(
        _final_body,
        out_shape=jax.ShapeDtypeStruct((1, 1), jnp.float32),
    )(x, aggx, deg, c, W1_self, W1_neigh, b1.reshape(1, H),
      W2_self, W2_neigh, b2.reshape(1, H), Wc, bc.reshape(1, 1))
    return pred.reshape(1)
